# baseline (speedup 1.0000x reference)
import jax
import jax.numpy as jnp
from jax import lax
from jax.experimental import pallas as pl
from jax.experimental.pallas import tpu as pltpu

_BF = jnp.bfloat16
_F32 = jnp.float32


def _full_spec(a):
    nd = a.ndim
    return pl.BlockSpec(a.shape, lambda b, _nd=nd: (0,) * _nd)


def _split(a):
    hi = a.astype(_BF)
    lo = (a - hi.astype(_F32)).astype(_BF)
    return hi, lo


def kernel(image_resnet, params, A, ref_vertices):
    B, D = image_resnet.shape
    n = ref_vertices.shape[0]
    npad = -(-n // 128) * 128

    A_pad = jnp.pad(A, ((0, npad - n), (0, npad - n)))
    refv = jnp.pad(ref_vertices, ((0, npad - n), (0, 0)))
    img3 = image_resnet.reshape(B, 1, D)

    args = []
    specs = []

    def add(a):
        args.append(a)
        specs.append(_full_spec(a))

    def add_split(a):
        hi, lo = _split(a)
        add(hi)
        add(lo)

    ih, il = _split(img3)
    args.append(ih)
    specs.append(pl.BlockSpec((1, 1, D), lambda b: (b, 0, 0)))
    args.append(il)
    specs.append(pl.BlockSpec((1, 1, D), lambda b: (b, 0, 0)))
    add_split(refv)
    add_split(A_pad)

    W0 = params["lin0"]["W"]
    add_split(W0[:, :3].T)
    add_split(W0[:, 3:].T)
    add(params["lin0"]["b"].reshape(1, -1))

    blocks = list(params["gc_blocks"]) + list(params["shape_blocks"])
    has_skip = []
    for p in blocks:
        add(p["pre_norm"]["gamma"].reshape(1, -1))
        add(p["pre_norm"]["beta"].reshape(1, -1))
        add_split(p["lin1"]["W"].T)
        add(p["lin1"]["b"].reshape(1, -1))
        add(p["norm1"]["gamma"].reshape(1, -1))
        add(p["norm1"]["beta"].reshape(1, -1))
        add_split(p["conv"]["W"])
        add(p["conv"]["b"].reshape(1, -1))
        add(p["norm2"]["gamma"].reshape(1, -1))
        add(p["norm2"]["beta"].reshape(1, -1))
        add_split(p["lin2"]["W"].T)
        add(p["lin2"]["b"].reshape(1, -1))
        hs = "skip" in p
        has_skip.append(hs)
        if hs:
            add_split(p["skip"]["W"].T)
            add(p["skip"]["b"].reshape(1, -1))

    add(params["final_gn"]["gamma"].reshape(1, -1))
    add(params["final_gn"]["beta"].reshape(1, -1))
    add_split(params["final_lin"]["W"])
    add(params["final_lin"]["b"].reshape(-1, 1))

    nf = float(n)

    def body(*refs):
        out_ref = refs[-1]
        it = iter(refs[:-1])

        def nxt():
            return next(it)[...]

        def nxt2():
            return nxt(), nxt()

        mask = (lax.broadcasted_iota(jnp.int32, (npad, 1), 0) < n
                ).astype(_F32)

        def d(u, v):
            return jnp.dot(u, v, preferred_element_type=_F32)

        def dot(a, w):
            wh, wl = w
            ah = a.astype(_BF)
            al = (a - ah.astype(_F32)).astype(_BF)
            return d(ah, wh) + d(ah, wl) + d(al, wh)

        def gn_relu(x, g, bb):
            C = x.shape[1]
            ii = lax.broadcasted_iota(jnp.int32, (C, C), 0) // 8
            jj = lax.broadcasted_iota(jnp.int32, (C, C), 1) // 8
            M = (ii == jj).astype(_BF)
            s = jnp.sum(x, axis=0, keepdims=True)
            s2 = jnp.sum(x * x, axis=0, keepdims=True)
            cnt = 8.0 * nf

            def gsum(v):
                vh = v.astype(_BF)
                vl = (v - vh.astype(_F32)).astype(_BF)
                return d(vh, M) + d(vl, M)

            mean = gsum(s) / cnt
            var = gsum(s2) / cnt - mean * mean
            sc = lax.rsqrt(var + 1e-5) * g
            sh = bb - mean * sc
            return jnp.maximum(x * sc + sh, 0.0) * mask

        img = nxt2()
        refw = nxt2()
        Aw = nxt2()
        w3t = nxt2()
        wimg = nxt2()
        b0 = nxt()

        def dot_pre(a, w):
            ah, al = a
            wh, wl = w
            return d(ah, wh) + d(ah, wl) + d(al, wh)

        x = (dot_pre(refw, w3t) + dot_pre((img[0][0], img[1][0]), wimg)
             + b0) * mask

        for hs in has_skip:
            gp, bp = nxt(), nxt()
            w1 = nxt2()
            b1 = nxt()
            g1, be1 = nxt(), nxt()
            wc = nxt2()
            bc = nxt()
            g2, be2 = nxt(), nxt()
            w2 = nxt2()
            b2 = nxt()
            y = gn_relu(x, gp, bp)
            y = (dot(y, w1) + b1) * mask
            y = gn_relu(y, g1, be1)
            sup = dot(y, wc)
            z = (dot_pre(Aw, _split_reg(sup)) + bc) * mask
            z = gn_relu(z, g2, be2)
            y2 = dot(z, w2) + b2
            if hs:
                ws = nxt2()
                bs = nxt()
                xs = dot(x, ws) + bs
            else:
                xs = x
            x = (xs + y2) * mask

        gf, bf = nxt(), nxt()
        wfh, wfl = nxt2()
        bfin = nxt()
        y = gn_relu(x, gf, bf)
        yh = y.astype(_BF)
        yl = (y - yh.astype(_F32)).astype(_BF)

        def dg(u, v):
            return lax.dot_general(u, v, (((1,), (1,)), ((), ())),
                                   preferred_element_type=_F32)

        outT = dg(wfh, yh) + dg(wfh, yl) + dg(wfl, yh)
        out_ref[0] = (outT + bfin)[:, :n]

    def _split_reg(a):
        hi = a.astype(_BF)
        lo = (a - hi.astype(_F32)).astype(_BF)
        return hi, lo

    out = pl.pallas_call(
        body,
        grid=(B,),
        in_specs=specs,
        out_specs=pl.BlockSpec((1, 3, n), lambda b: (b, 0, 0)),
        out_shape=jax.ShapeDtypeStruct((B, 3, n), _F32),
        compiler_params=pltpu.CompilerParams(
            dimension_semantics=("parallel",)),
    )(*args)
    return out


# speedup vs baseline: 2.1892x; 2.1892x over previous
import jax
import jax.numpy as jnp
from jax import lax
from jax.experimental import pallas as pl
from jax.experimental.pallas import tpu as pltpu

_BF = jnp.bfloat16
_F32 = jnp.float32


def _full_spec(a):
    nd = a.ndim
    return pl.BlockSpec(a.shape, lambda b, _nd=nd: (0,) * _nd)


def kernel(image_resnet, params, A, ref_vertices):
    B, D = image_resnet.shape
    n = ref_vertices.shape[0]
    npad = -(-n // 128) * 128

    A_pad = jnp.pad(A, ((0, npad - n), (0, npad - n)))
    refv = jnp.pad(ref_vertices, ((0, npad - n), (0, 0)))
    img3 = image_resnet.reshape(B, 1, D).astype(_BF)

    args = [img3]
    specs = [pl.BlockSpec((1, 1, D), lambda b: (b, 0, 0))]

    def add(a):
        args.append(a)
        specs.append(_full_spec(a))

    add(refv.astype(_BF))
    add(A_pad.astype(_BF))

    W0 = params["lin0"]["W"]
    add(W0[:, :3].T.astype(_BF))
    add(W0[:, 3:].T.astype(_BF))
    add(params["lin0"]["b"].reshape(1, -1))

    blocks = list(params["gc_blocks"]) + list(params["shape_blocks"])
    has_skip = []
    for p in blocks:
        add(p["pre_norm"]["gamma"].reshape(1, -1))
        add(p["pre_norm"]["beta"].reshape(1, -1))
        add(p["lin1"]["W"].T.astype(_BF))
        add(p["lin1"]["b"].reshape(1, -1))
        add(p["norm1"]["gamma"].reshape(1, -1))
        add(p["norm1"]["beta"].reshape(1, -1))
        add(p["conv"]["W"].astype(_BF))
        add(p["conv"]["b"].reshape(1, -1))
        add(p["norm2"]["gamma"].reshape(1, -1))
        add(p["norm2"]["beta"].reshape(1, -1))
        add(p["lin2"]["W"].T.astype(_BF))
        add(p["lin2"]["b"].reshape(1, -1))
        hs = "skip" in p
        has_skip.append(hs)
        if hs:
            add(p["skip"]["W"].T.astype(_BF))
            add(p["skip"]["b"].reshape(1, -1))

    add(params["final_gn"]["gamma"].reshape(1, -1))
    add(params["final_gn"]["beta"].reshape(1, -1))
    add(params["final_lin"]["W"].astype(_BF))
    add(params["final_lin"]["b"].reshape(-1, 1))

    nf = float(n)

    def body(*refs):
        out_ref = refs[-1]
        it = iter(refs[:-1])

        def nxt():
            return next(it)[...]

        mask = (lax.broadcasted_iota(jnp.int32, (npad, 1), 0) < n
                ).astype(_F32)

        def d(u, v):
            return jnp.dot(u, v, preferred_element_type=_F32)

        def dot(a, w):
            return d(a.astype(_BF), w)

        def gn_relu(x, g, bb):
            C = x.shape[1]
            ii = lax.broadcasted_iota(jnp.int32, (C, C), 0) // 8
            jj = lax.broadcasted_iota(jnp.int32, (C, C), 1) // 8
            M = (ii == jj).astype(_BF)
            s = jnp.sum(x, axis=0, keepdims=True)
            s2 = jnp.sum(x * x, axis=0, keepdims=True)
            cnt = 8.0 * nf

            def gsum(v):
                vh = v.astype(_BF)
                vl = (v - vh.astype(_F32)).astype(_BF)
                return d(vh, M) + d(vl, M)

            mean = gsum(s) / cnt
            var = gsum(s2) / cnt - mean * mean
            sc = lax.rsqrt(var + 1e-5) * g
            sh = bb - mean * sc
            return jnp.maximum(x * sc + sh, 0.0) * mask

        img = nxt()
        refw = nxt()
        Ab = nxt()
        w3t = nxt()
        wimg = nxt()
        b0 = nxt()

        x = (d(refw, w3t) + d(img[0], wimg) + b0) * mask

        for hs in has_skip:
            gp, bp = nxt(), nxt()
            w1, b1 = nxt(), nxt()
            g1, be1 = nxt(), nxt()
            wc, bc = nxt(), nxt()
            g2, be2 = nxt(), nxt()
            w2, b2 = nxt(), nxt()
            y = gn_relu(x, gp, bp)
            y = (dot(y, w1) + b1) * mask
            y = gn_relu(y, g1, be1)
            sup = dot(y, wc)
            z = (d(Ab, sup.astype(_BF)) + bc) * mask
            z = gn_relu(z, g2, be2)
            y2 = dot(z, w2) + b2
            if hs:
                ws, bs = nxt(), nxt()
                xs = dot(x, ws) + bs
            else:
                xs = x
            x = (xs + y2) * mask

        gf, bf = nxt(), nxt()
        wf, bfin = nxt(), nxt()
        y = gn_relu(x, gf, bf)
        outT = lax.dot_general(wf, y.astype(_BF), (((1,), (1,)), ((), ())),
                               preferred_element_type=_F32)
        out_ref[0] = (outT + bfin)[:, :n]

    out = pl.pallas_call(
        body,
        grid=(B,),
        in_specs=specs,
        out_specs=pl.BlockSpec((1, 3, n), lambda b: (b, 0, 0)),
        out_shape=jax.ShapeDtypeStruct((B, 3, n), _F32),
        compiler_params=pltpu.CompilerParams(
            dimension_semantics=("parallel",)),
    )(*args)
    return out
